# baseline (speedup 1.0000x reference)
import functools

import jax
import jax.numpy as jnp
from jax import lax
from jax.experimental import pallas as pl
from jax.experimental.pallas import tpu as pltpu
from jax.experimental.pallas import tpu_sc as plsc

RC = 5.2
OUTPUT_SIZE = 16

N_BATCH = 100
N_CONN = 32000
N_ATOMS = 1000

NC, NS, L = 2, 16, 16
NW = NC * NS
EB = 128
N_BLOCKS = N_CONN // EB
BLOCKS_PER_W = -(-N_BLOCKS // NW)
GROUPS_PER_B = EB // L
CPAD = 3072

_INV_RC = 1.0 / RC
_PI = 3.14159265358979

_S3 = -1.0 / 6.0
_S5 = 1.0 / 120.0
_S7 = -1.0 / 5040.0
_S9 = 1.0 / 362880.0


def _sc_kernel(conn_hbm, coords_hbm, params_hbm, out_hbm,
               conn_v, coords_v, params_v, out_v,
               sem_conn, sem_out, sem_coords):
    wid = lax.axis_index("s") * NC + lax.axis_index("c")

    pltpu.sync_copy(params_hbm, params_v)

    neg_eta = params_v[pl.ds(OUTPUT_SIZE * L, L)]
    shells = [params_v[pl.ds(j * L, L)] for j in range(OUTPUT_SIZE)]

    def coords_src(b):
        return coords_hbm.at[pl.ds(b * (N_ATOMS * 3), N_ATOMS * 3)]

    def coords_dst(slot):
        return coords_v.at[pl.ds(slot * CPAD, N_ATOMS * 3)]

    pltpu.async_copy(coords_src(0), coords_dst(0), sem_coords)

    def batch_body(b, carry):
        slot = b % 2
        pltpu.make_async_copy(coords_src(b), coords_dst(slot),
                              sem_coords).wait()

        @pl.when(b + 1 < N_BATCH)
        def _():
            pltpu.async_copy(coords_src(b + 1), coords_dst(1 - slot),
                             sem_coords)

        soff = jnp.zeros((L,), jnp.int32) + slot * CPAD

        def conn_src(t):
            return conn_hbm.at[b, pl.ds(0, 2),
                               pl.ds((t * NW + wid) * EB, EB)]

        def out_dst(t):
            return out_hbm.at[b, pl.ds(0, OUTPUT_SIZE),
                              pl.ds((t * NW + wid) * EB, EB)]

        pltpu.async_copy(conn_src(0), conn_v.at[0], sem_conn)

        for t in range(BLOCKS_PER_W):
            tb = t % 2
            if t + 1 < BLOCKS_PER_W:
                @pl.when((t + 1) * NW + wid < N_BLOCKS)
                def _(t=t):
                    pltpu.async_copy(conn_src(t + 1), conn_v.at[(t + 1) % 2],
                                     sem_conn)

            @pl.when(t * NW + wid < N_BLOCKS)
            def _(t=t, tb=tb):
                pltpu.make_async_copy(conn_src(t), conn_v.at[tb],
                                      sem_conn).wait()
                if t >= 2:
                    pltpu.make_async_copy(out_v.at[tb], out_dst(t - 2),
                                          sem_out).wait()
                for g in range(GROUPS_PER_B):
                    ia = conn_v[tb, 0, pl.ds(g * L, L)]
                    idn = conn_v[tb, 1, pl.ds(g * L, L)]
                    ia3 = ia * 3 + soff
                    id3 = idn * 3 + soff
                    xa = plsc.load_gather(coords_v, [ia3])
                    ya = plsc.load_gather(coords_v, [ia3 + 1])
                    za = plsc.load_gather(coords_v, [ia3 + 2])
                    xd = plsc.load_gather(coords_v, [id3])
                    yd = plsc.load_gather(coords_v, [id3 + 1])
                    zd = plsc.load_gather(coords_v, [id3 + 2])
                    dx = xa - xd
                    dy = ya - yd
                    dz = za - zd
                    r2 = dx * dx + dy * dy + dz * dz
                    r2s = jnp.maximum(r2, 1e-24)
                    bits = plsc.bitcast(r2s, jnp.int32)
                    y = plsc.bitcast(jnp.int32(0x5F3759DF) - (bits >> 1),
                                     jnp.float32)
                    h = 0.5 * r2s
                    y = y * (1.5 - h * y * y)
                    y = y * (1.5 - h * y * y)
                    y = y * (1.5 - h * y * y)
                    d = r2 * y
                    u = jnp.minimum(d * _INV_RC, 1.0)
                    tt = (u - 0.5) * _PI
                    t2 = tt * tt
                    p = 1.0 + t2 * (_S3 + t2 * (_S5 + t2 * (_S7 + t2 * _S9)))
                    fcq = 0.125 + (tt * -0.125) * p
                    for j in range(OUTPUT_SIZE):
                        tj = d - shells[j]
                        ej = jnp.exp(neg_eta * (tj * tj))
                        out_v[tb, j, pl.ds(g * L, L)] = ej * fcq
                pltpu.async_copy(out_v.at[tb], out_dst(t), sem_out)

        for t in (BLOCKS_PER_W - 2, BLOCKS_PER_W - 1):
            @pl.when(t * NW + wid < N_BLOCKS)
            def _(t=t):
                pltpu.make_async_copy(out_v.at[t % 2], out_dst(t),
                                      sem_out).wait()
        return carry

    lax.fori_loop(0, N_BATCH, batch_body, 0, unroll=False)


@jax.jit
def _apev(connectivity, coords, EtaR, ShfR):
    conn2 = jnp.swapaxes(connectivity.astype(jnp.int32), 1, 2)
    coords2 = coords.reshape(N_BATCH * N_ATOMS * 3)
    shf_splat = jnp.repeat(ShfR.astype(jnp.float32), L)
    eta_splat = jnp.broadcast_to(-EtaR.astype(jnp.float32), (L,))
    params = jnp.concatenate([shf_splat, eta_splat])
    mesh = plsc.VectorSubcoreMesh(core_axis_name="c", subcore_axis_name="s")
    run = pl.kernel(
        _sc_kernel,
        out_type=jax.ShapeDtypeStruct((N_BATCH, OUTPUT_SIZE, N_CONN),
                                      jnp.float32),
        mesh=mesh,
        compiler_params=pltpu.CompilerParams(needs_layout_passes=False),
        scratch_types=[
            pltpu.VMEM((2, 2, EB), jnp.int32),
            pltpu.VMEM((2 * CPAD,), jnp.float32),
            pltpu.VMEM(((OUTPUT_SIZE + 1) * L,), jnp.float32),
            pltpu.VMEM((2, OUTPUT_SIZE, EB), jnp.float32),
            pltpu.SemaphoreType.DMA,
            pltpu.SemaphoreType.DMA,
            pltpu.SemaphoreType.DMA,
        ],
    )
    yt = run(conn2, coords2, params)
    return jnp.swapaxes(yt, 1, 2)


def kernel(connectivity, coords, EtaR, ShfR):
    y = _apev(connectivity, coords, EtaR, ShfR)
    return (connectivity, y)


# speedup vs baseline: 2.0608x; 2.0608x over previous
import functools

import jax
import jax.numpy as jnp
from jax import lax
from jax.experimental import pallas as pl
from jax.experimental.pallas import tpu as pltpu
from jax.experimental.pallas import tpu_sc as plsc

RC = 5.2
OUTPUT_SIZE = 16

N_BATCH = 100
N_CONN = 32000
N_ATOMS = 1000

NC, NS, L = 2, 16, 16
NW = NC * NS
EB = 128
N_BLOCKS = N_CONN // EB
BLOCKS_PER_W = 8
SPAN = BLOCKS_PER_W * EB
GROUPS_PER_B = EB // L

_INV_RC = 1.0 / RC
_PI = 3.14159265358979

_S3 = -1.0 / 6.0
_S5 = 1.0 / 120.0
_S7 = -1.0 / 5040.0
_S9 = 1.0 / 362880.0


def _sc_kernel(conn_hbm, coords_hbm, params_hbm, out_hbm,
               conn_v0, conn_v1, coords_v0, coords_v1, params_v,
               out_v0, out_v1, sem_conn, sem_out, sem_coords):
    wid = lax.axis_index("s") * NC + lax.axis_index("c")
    estart = jnp.minimum(wid * SPAN, N_CONN - SPAN)

    pltpu.sync_copy(params_hbm, params_v)

    neg_eta = params_v[pl.ds(OUTPUT_SIZE * L, L)]
    shells = [params_v[pl.ds(j * L, L)] for j in range(OUTPUT_SIZE)]

    def coords_src(b):
        return coords_hbm.at[pl.ds(b * (N_ATOMS * 3), N_ATOMS * 3)]

    def conn_src(b):
        return conn_hbm.at[b, pl.ds(0, 2), pl.ds(estart, SPAN)]

    def out_dst(b):
        return out_hbm.at[b, pl.ds(0, OUTPUT_SIZE), pl.ds(estart, SPAN)]

    pltpu.async_copy(coords_src(0), coords_v0, sem_coords)
    pltpu.async_copy(conn_src(0), conn_v0, sem_conn)
    pltpu.async_copy(coords_src(1), coords_v1, sem_coords)
    pltpu.async_copy(conn_src(1), conn_v1, sem_conn)

    def half(k, b, conn_v, coords_v, out_v):
        pltpu.make_async_copy(coords_src(b), coords_v, sem_coords).wait()
        pltpu.make_async_copy(conn_src(b), conn_v, sem_conn).wait()

        @pl.when(k > 0)
        def _():
            pltpu.make_async_copy(out_v, out_dst(b - 2), sem_out).wait()

        def block_body(t, carry):
            toff = t * EB
            for g in range(GROUPS_PER_B):
                ia = conn_v[0, pl.ds(toff + g * L, L)]
                idn = conn_v[1, pl.ds(toff + g * L, L)]
                ia3 = ia * 3
                id3 = idn * 3
                xa = plsc.load_gather(coords_v, [ia3])
                ya = plsc.load_gather(coords_v, [ia3 + 1])
                za = plsc.load_gather(coords_v, [ia3 + 2])
                xd = plsc.load_gather(coords_v, [id3])
                yd = plsc.load_gather(coords_v, [id3 + 1])
                zd = plsc.load_gather(coords_v, [id3 + 2])
                dx = xa - xd
                dy = ya - yd
                dz = za - zd
                r2 = dx * dx + dy * dy + dz * dz
                r2s = jnp.maximum(r2, 1e-24)
                bits = plsc.bitcast(r2s, jnp.int32)
                y = plsc.bitcast(jnp.int32(0x5F3759DF) - (bits >> 1),
                                 jnp.float32)
                h = 0.5 * r2s
                y = y * (1.5 - h * y * y)
                y = y * (1.5 - h * y * y)
                y = y * (1.5 - h * y * y)
                d = r2 * y
                u = jnp.minimum(d * _INV_RC, 1.0)
                tt = (u - 0.5) * _PI
                t2 = tt * tt
                p = 1.0 + t2 * (_S3 + t2 * (_S5 + t2 * (_S7 + t2 * _S9)))
                fcq = 0.125 + (tt * -0.125) * p
                for j in range(OUTPUT_SIZE):
                    tj = d - shells[j]
                    ej = jnp.exp(neg_eta * (tj * tj))
                    out_v[j, pl.ds(toff + g * L, L)] = ej * fcq
            return carry

        lax.fori_loop(0, BLOCKS_PER_W, block_body, 0, unroll=False)
        pltpu.async_copy(out_v, out_dst(b), sem_out)

        @pl.when(b + 2 < N_BATCH)
        def _():
            pltpu.async_copy(coords_src(b + 2), coords_v, sem_coords)
            pltpu.async_copy(conn_src(b + 2), conn_v, sem_conn)

    def batch_body(k, carry):
        half(k, 2 * k, conn_v0, coords_v0, out_v0)
        half(k, 2 * k + 1, conn_v1, coords_v1, out_v1)
        return carry

    lax.fori_loop(0, N_BATCH // 2, batch_body, 0, unroll=False)
    pltpu.make_async_copy(out_v0, out_dst(N_BATCH - 2), sem_out).wait()
    pltpu.make_async_copy(out_v1, out_dst(N_BATCH - 1), sem_out).wait()


@jax.jit
def _apev(connectivity, coords, EtaR, ShfR):
    conn2 = jnp.swapaxes(connectivity.astype(jnp.int32), 1, 2)
    coords2 = coords.reshape(N_BATCH * N_ATOMS * 3)
    shf_splat = jnp.repeat(ShfR.astype(jnp.float32), L)
    eta_splat = jnp.broadcast_to(-EtaR.astype(jnp.float32), (L,))
    params = jnp.concatenate([shf_splat, eta_splat])
    mesh = plsc.VectorSubcoreMesh(core_axis_name="c", subcore_axis_name="s")
    run = pl.kernel(
        _sc_kernel,
        out_type=jax.ShapeDtypeStruct((N_BATCH, OUTPUT_SIZE, N_CONN),
                                      jnp.float32),
        mesh=mesh,
        compiler_params=pltpu.CompilerParams(needs_layout_passes=False),
        scratch_types=[
            pltpu.VMEM((2, SPAN), jnp.int32),
            pltpu.VMEM((2, SPAN), jnp.int32),
            pltpu.VMEM((N_ATOMS * 3,), jnp.float32),
            pltpu.VMEM((N_ATOMS * 3,), jnp.float32),
            pltpu.VMEM(((OUTPUT_SIZE + 1) * L,), jnp.float32),
            pltpu.VMEM((OUTPUT_SIZE, SPAN), jnp.float32),
            pltpu.VMEM((OUTPUT_SIZE, SPAN), jnp.float32),
            pltpu.SemaphoreType.DMA,
            pltpu.SemaphoreType.DMA,
            pltpu.SemaphoreType.DMA,
        ],
    )
    yt = run(conn2, coords2, params)
    return jnp.swapaxes(yt, 1, 2)


def kernel(connectivity, coords, EtaR, ShfR):
    y = _apev(connectivity, coords, EtaR, ShfR)
    return (connectivity, y)
